# baseline (speedup 1.0000x reference)
import functools

import jax
import jax.numpy as jnp
from jax import lax
from jax.experimental import pallas as pl
from jax.experimental.pallas import tpu as pltpu
from jax.experimental.pallas import tpu_sc as plsc


def _tc_tableprep(weight_int8):
    V, D = weight_int8.shape
    W = D // 4
    R = 8000

    def prep(w_ref, out_ref):
        x = w_ref[...].astype(jnp.int32)
        b0 = x[:, 0 * W:1 * W] & 255
        b1 = x[:, 1 * W:2 * W] & 255
        b2 = x[:, 2 * W:3 * W] & 255
        b3 = x[:, 3 * W:4 * W]
        out_ref[...] = (b0 + (b1 << 8)) + ((b2 << 16) + (b3 << 24))

    return pl.pallas_call(
        prep,
        grid=(V // R,),
        in_specs=[pl.BlockSpec((R, D), lambda i: (i, 0))],
        out_specs=pl.BlockSpec((R, W), lambda i: (i, 0)),
        out_shape=jax.ShapeDtypeStruct((V, W), jnp.int32),
    )(weight_int8)


def _sc_gather(idx, tab_i32, weight_scale):
    BL = idx.shape[0]
    V, W = tab_i32.shape
    NC, NS = 2, 16
    NW = NC * NS
    per_w = BL // NW
    C = 2560
    n_chunks = per_w // C

    mesh = plsc.VectorSubcoreMesh(core_axis_name="c", subcore_axis_name="s")

    @functools.partial(
        pl.kernel,
        mesh=mesh,
        compiler_params=pltpu.CompilerParams(use_tc_tiling_on_sc=False),
        out_type=[
            jax.ShapeDtypeStruct((BL, W), jnp.int32),
            jax.ShapeDtypeStruct((BL,), jnp.float32),
        ],
        scratch_types=[
            pltpu.VMEM((per_w,), jnp.int32),
            pltpu.VMEM((C, W), jnp.int32),
            pltpu.VMEM((C,), jnp.float32),
            pltpu.SemaphoreType.DMA,
            pltpu.SemaphoreType.DMA,
        ],
    )
    def gather_k(idx_hbm, tab_hbm, scale_hbm, rows_out, scale_out,
                 idx_v, rows_v, sc_v, sem_r, sem_s):
        wid = lax.axis_index("s") * NC + lax.axis_index("c")
        base = pl.multiple_of(wid * per_w, 8)
        pltpu.sync_copy(idx_hbm.at[pl.ds(base, per_w)], idx_v)

        def body(g, carry):
            off = pl.multiple_of(g * C, 8)
            idx_c = idx_v.at[pl.ds(off, C)]
            cp_r = pltpu.async_copy(tab_hbm.at[idx_c], rows_v, sem_r)
            cp_s = pltpu.async_copy(scale_hbm.at[idx_c], sc_v, sem_s)
            cp_r.wait()
            cp_s.wait()
            dst = pl.multiple_of(base + off, 8)
            pltpu.sync_copy(rows_v, rows_out.at[pl.ds(dst, C)])
            pltpu.sync_copy(sc_v, scale_out.at[pl.ds(dst, C)])
            return carry

        lax.fori_loop(0, n_chunks, body, 0)

    return gather_k(idx, tab_i32, weight_scale)


def _tc_dequant(rows_w, scales, B, L, D):
    GB = 64
    TR = GB * L
    W = D // 4

    def dq(rows_ref, scale_ref, out_ref):
        x = rows_ref[...]
        planes = [
            (x << 24) >> 24,
            (x << 16) >> 24,
            (x << 8) >> 24,
            x >> 24,
        ]
        y = jnp.concatenate(planes, axis=1).astype(jnp.float32)
        y = (y * scale_ref[...]).astype(jnp.bfloat16)
        out_ref[...] = y.reshape(GB, L, D)

    return pl.pallas_call(
        dq,
        grid=(B // GB,),
        in_specs=[
            pl.BlockSpec((TR, W), lambda i: (i, 0)),
            pl.BlockSpec((TR, 1), lambda i: (i, 0)),
        ],
        out_specs=pl.BlockSpec((GB, L, D), lambda i: (i, 0, 0)),
        out_shape=jax.ShapeDtypeStruct((B, L, D), jnp.bfloat16),
    )(rows_w, scales)


def kernel(input, weight_int8, weight_scale):
    B, L = input.shape
    V, D = weight_int8.shape
    BL = B * L
    tab_i32 = _tc_tableprep(weight_int8)
    rows_w, scale_w = _sc_gather(input.reshape(BL), tab_i32, weight_scale)
    return _tc_dequant(rows_w, scale_w.reshape(BL, 1), B, L, D)


# speedup vs baseline: 1.0350x; 1.0350x over previous
import functools

import jax
import jax.numpy as jnp
from jax import lax
from jax.experimental import pallas as pl
from jax.experimental.pallas import tpu as pltpu
from jax.experimental.pallas import tpu_sc as plsc


def _tc_tableprep(weight_int8):
    V, D = weight_int8.shape
    W = D // 4
    R = 8000

    def prep(w_ref, out_ref):
        x = w_ref[...].astype(jnp.int32)
        b0 = x[:, 0 * W:1 * W] & 255
        b1 = x[:, 1 * W:2 * W] & 255
        b2 = x[:, 2 * W:3 * W] & 255
        b3 = x[:, 3 * W:4 * W]
        out_ref[...] = (b0 + (b1 << 8)) + ((b2 << 16) + (b3 << 24))

    return pl.pallas_call(
        prep,
        grid=(V // R,),
        in_specs=[pl.BlockSpec((R, D), lambda i: (i, 0))],
        out_specs=pl.BlockSpec((R, W), lambda i: (i, 0)),
        out_shape=jax.ShapeDtypeStruct((V, W), jnp.int32),
    )(weight_int8)


def _sc_gather(idx, tab_i32, weight_scale):
    BL = idx.shape[0]
    V, W = tab_i32.shape
    NC, NS = 2, 16
    NW = NC * NS
    per_w = BL // NW
    C = 2560
    n_chunks = per_w // C

    mesh = plsc.VectorSubcoreMesh(core_axis_name="c", subcore_axis_name="s")

    @functools.partial(
        pl.kernel,
        mesh=mesh,
        compiler_params=pltpu.CompilerParams(use_tc_tiling_on_sc=False),
        out_type=[
            jax.ShapeDtypeStruct((BL, W), jnp.int32),
            jax.ShapeDtypeStruct((BL,), jnp.float32),
        ],
        scratch_types=[
            pltpu.VMEM((per_w,), jnp.int32),
            pltpu.VMEM((C, W), jnp.int32),
            pltpu.VMEM((C,), jnp.float32),
            pltpu.SemaphoreType.DMA,
            pltpu.SemaphoreType.DMA,
        ],
    )
    def gather_k(idx_hbm, tab_hbm, scale_hbm, rows_out, scale_out,
                 idx_v, rows_v, sc_v, sem_r, sem_s):
        wid = lax.axis_index("s") * NC + lax.axis_index("c")
        base = pl.multiple_of(wid * per_w, 8)
        pltpu.sync_copy(idx_hbm.at[pl.ds(base, per_w)], idx_v)

        def body(g, carry):
            off = pl.multiple_of(g * C, 8)
            idx_c = idx_v.at[pl.ds(off, C)]
            cp_r = pltpu.async_copy(tab_hbm.at[idx_c], rows_v, sem_r)
            cp_s = pltpu.async_copy(scale_hbm.at[idx_c], sc_v, sem_s)
            cp_r.wait()
            cp_s.wait()
            dst = pl.multiple_of(base + off, 8)
            pltpu.sync_copy(rows_v, rows_out.at[pl.ds(dst, C)])
            pltpu.sync_copy(sc_v, scale_out.at[pl.ds(dst, C)])
            return carry

        lax.fori_loop(0, n_chunks, body, 0)

    return gather_k(idx, tab_i32, weight_scale)


def _tc_dequant(rows2, scales, B, L, D):
    GB = 64
    TR = GB * L
    W = D // 4
    WR = TR * W // 128

    def dq(rows_ref, scale_ref, out_ref):
        x = rows_ref[...]
        planes = [
            (x << 24) >> 24,
            (x << 16) >> 24,
            (x << 8) >> 24,
            x >> 24,
        ]
        parts = []
        for q in range(8):
            sub = [p[:, 16 * q:16 * (q + 1)] for p in planes]
            parts.append(jnp.concatenate(sub, axis=1)[:, None, :])
        y = jnp.concatenate(parts, axis=1).reshape(TR, D)
        y = (y.astype(jnp.float32) * scale_ref[...]).astype(jnp.bfloat16)
        out_ref[...] = y.reshape(GB, L, D)

    return pl.pallas_call(
        dq,
        grid=(B // GB,),
        in_specs=[
            pl.BlockSpec((WR, 128), lambda i: (i, 0)),
            pl.BlockSpec((TR, 1), lambda i: (i, 0)),
        ],
        out_specs=pl.BlockSpec((GB, L, D), lambda i: (i, 0, 0)),
        out_shape=jax.ShapeDtypeStruct((B, L, D), jnp.bfloat16),
    )(rows2, scales)


def kernel(input, weight_int8, weight_scale):
    B, L = input.shape
    V, D = weight_int8.shape
    BL = B * L
    tab_i32 = _tc_tableprep(weight_int8)
    rows_w, scale_w = _sc_gather(input.reshape(BL), tab_i32, weight_scale)
    rows2 = rows_w.reshape(BL * (D // 4) // 128, 128)
    return _tc_dequant(rows2, scale_w.reshape(BL, 1), B, L, D)
